# baseline (speedup 1.0000x reference)
import functools

import jax
import jax.numpy as jnp
from jax import lax
from jax.experimental import pallas as pl
from jax.experimental.pallas import tpu as pltpu
from jax.experimental.pallas import tpu_sc as plsc

_BSC = 128
_TCB = 200


def _precompute_body(idx_ref, tok_ref, pos_ref, w_ref, b_ref,
                     e_ref, ebf_ref, p_ref, pexp_ref, pt_ref, idx3_ref):
    dn = (((1,), (1,)), ((), ()))
    e = lax.dot_general(tok_ref[...], w_ref[...], dn,
                        preferred_element_type=jnp.float32)
    e_ref[...] = e
    ebf_ref[...] = e.astype(jnp.bfloat16)
    p = lax.dot_general(pos_ref[...], w_ref[...], dn,
                        preferred_element_type=jnp.float32) + b_ref[...]
    p_ref[...] = p
    T, V = p.shape
    reps = _TCB // T
    pexp_ref[...] = jnp.concatenate([p] * reps, axis=0)
    rem = V - 16 * (V // 16)
    pt_ref[...] = jnp.concatenate(
        [jnp.zeros((T, 16 - rem), jnp.float32), p[:, V - rem:]], axis=1)
    HT = T // 2
    idx = idx_ref[...]
    zeros7 = jnp.zeros((_BSC, 32 - HT), jnp.int32)
    idx3_ref[0, :, :] = jnp.concatenate([idx[:_BSC, :HT], zeros7], axis=1)
    idx3_ref[1, :, :] = jnp.concatenate([idx[:_BSC, HT:], zeros7], axis=1)


def _precompute(indices, tok_table, pos_t, W, b2d):
    V, _ = W.shape
    T = pos_t.shape[0]
    return pl.pallas_call(
        _precompute_body,
        out_shape=[
            jax.ShapeDtypeStruct((V, V), jnp.float32),
            jax.ShapeDtypeStruct((V, V), jnp.bfloat16),
            jax.ShapeDtypeStruct((T, V), jnp.float32),
            jax.ShapeDtypeStruct((_TCB, V), jnp.float32),
            jax.ShapeDtypeStruct((T, 16), jnp.float32),
            jax.ShapeDtypeStruct((2, _BSC, 32), jnp.int32),
        ],
    )(indices, tok_table, pos_t, W, b2d)


def _make_sc_gather(V, T, B):
    info = plsc.get_sparse_core_info()
    NC, NS = info.num_cores, info.num_subcores
    HT = T // 2
    CP = 32
    BW = _BSC // (NC * NS // 2)
    NCH = BW
    n_lane = V // 16

    mesh = plsc.VectorSubcoreMesh(core_axis_name="c", subcore_axis_name="s")

    @functools.partial(
        pl.kernel,
        mesh=mesh,
        out_type=jax.ShapeDtypeStruct((B * T, V), jnp.float32),
        scratch_types=[
            pltpu.VMEM((NCH, CP), jnp.int32),
            pltpu.VMEM((CP, V), jnp.float32),
            pltpu.VMEM((HT, V), jnp.float32),
            pltpu.VMEM((HT, 16), jnp.float32),
            pltpu.SemaphoreType.DMA,
            pltpu.SemaphoreType.DMA,
        ],
        compiler_params=pltpu.CompilerParams(use_tc_tiling_on_sc=False),
    )
    def gather_kernel(idx_hbm, e_hbm, p_hbm, pt_hbm, out_hbm,
                      idx_v, buf0, p_v, pt_v, g0, s0):
        wid = lax.axis_index("s") * NC + lax.axis_index("c")
        wb = wid // 2
        wt = wid % 2
        pltpu.sync_copy(idx_hbm.at[wt, pl.ds(wb * BW, BW), :], idx_v)
        pltpu.sync_copy(p_hbm.at[pl.ds(wt * HT, HT), :], p_v)
        pltpu.sync_copy(pt_hbm.at[pl.ds(wt * HT, HT), :], pt_v)

        def gather(j):
            return pltpu.make_async_copy(
                e_hbm.at[idx_v.at[j, :]], buf0, g0)

        def scatter(j):
            row0 = (wb * BW + j) * T + wt * HT
            return pltpu.make_async_copy(
                buf0.at[pl.ds(0, HT), :],
                out_hbm.at[pl.ds(row0, HT), :], s0)

        def add_p(buf):
            @plsc.parallel_loop(0, HT, unroll=2)
            def _(i):
                for j in range(n_lane):
                    sl = pl.ds(j * 16, 16)
                    buf[i, sl] = buf[i, sl] + p_v[i, sl]
                sl = pl.ds(V - 16, 16)
                buf[i, sl] = buf[i, sl] + pt_v[i, :]

        def body(k, carry):
            gather(k).start()
            gather(k).wait()
            add_p(buf0)
            scatter(k).start()
            scatter(k).wait()
            return carry

        lax.fori_loop(0, NCH, body, 0)

    return gather_kernel


def _onehot_body(idx_ref, ebf_ref, pexp_ref, sc_ref, out_ref):
    del sc_ref
    idxv = idx_ref[0]
    V = ebf_ref.shape[0]
    iot = lax.broadcasted_iota(jnp.int32, (V, _TCB), 0)
    oht = (iot == idxv).astype(jnp.bfloat16)
    out_ref[...] = lax.dot_general(
        oht, ebf_ref[...], (((0,), (0,)), ((), ())),
        preferred_element_type=jnp.float32) + pexp_ref[...]


def _tc_onehot(idx2d, Ebf, Pexp, sc_out, BT, V):
    n_sc_blocks = _BSC * 50 // _TCB
    n_blocks = BT // _TCB - n_sc_blocks
    return pl.pallas_call(
        _onehot_body,
        grid=(n_blocks,),
        in_specs=[
            pl.BlockSpec((1, 1, _TCB), lambda g: (g + n_sc_blocks, 0, 0)),
            pl.BlockSpec((V, V), lambda g: (0, 0)),
            pl.BlockSpec((_TCB, V), lambda g: (0, 0)),
            pl.BlockSpec(memory_space=pl.ANY),
        ],
        out_specs=pl.BlockSpec((_TCB, V), lambda g: (g + n_sc_blocks, 0)),
        out_shape=jax.ShapeDtypeStruct((BT, V), jnp.float32),
        input_output_aliases={3: 0},
    )(idx2d, Ebf, Pexp, sc_out)


def kernel(indices, tok_table, pos_table, W, b):
    Bsz, T = indices.shape
    V = W.shape[0]
    BT = Bsz * T

    idx32 = indices.astype(jnp.int32)
    E, Ebf, P, Pexp, PT, IDX3 = _precompute(idx32, tok_table,
                                            pos_table[:T], W,
                                            b.reshape(1, V))
    sc_out = _make_sc_gather(V, T, Bsz)(IDX3, E, P, PT)
    idx2d = idx32.reshape(BT // _TCB, 1, _TCB)
    out = _tc_onehot(idx2d, Ebf, Pexp, sc_out, BT, V)
    return out.reshape(Bsz, T, V)


# speedup vs baseline: 1.5034x; 1.5034x over previous
import functools

import jax
import jax.numpy as jnp
from jax import lax
from jax.experimental import pallas as pl
from jax.experimental.pallas import tpu as pltpu
from jax.experimental.pallas import tpu_sc as plsc

_BSC = 64
_TCB = 200


def _precompute_body(idx_ref, tok_ref, pos_ref, w_ref, b_ref,
                     e_ref, ebf_ref, p_ref, pexp_ref, pt_ref, idx3_ref):
    dn = (((1,), (1,)), ((), ()))
    e = lax.dot_general(tok_ref[...], w_ref[...], dn,
                        preferred_element_type=jnp.float32)
    e_ref[...] = e
    ebf_ref[...] = e.astype(jnp.bfloat16)
    p = lax.dot_general(pos_ref[...], w_ref[...], dn,
                        preferred_element_type=jnp.float32) + b_ref[...]
    p_ref[...] = p
    T, V = p.shape
    reps = _TCB // T
    pexp_ref[...] = jnp.concatenate([p] * reps, axis=0)
    rem = V - 16 * (V // 16)
    pt_ref[...] = jnp.concatenate(
        [jnp.zeros((T, 16 - rem), jnp.float32), p[:, V - rem:]], axis=1)
    HT = T // 2
    idx = idx_ref[...]
    zeros7 = jnp.zeros((_BSC, 32 - HT), jnp.int32)
    idx3_ref[0, :, :] = jnp.concatenate([idx[:_BSC, :HT], zeros7], axis=1)
    idx3_ref[1, :, :] = jnp.concatenate([idx[:_BSC, HT:], zeros7], axis=1)


def _precompute(indices, tok_table, pos_t, W, b2d):
    V, _ = W.shape
    T = pos_t.shape[0]
    return pl.pallas_call(
        _precompute_body,
        out_shape=[
            jax.ShapeDtypeStruct((V, V), jnp.float32),
            jax.ShapeDtypeStruct((V, V), jnp.bfloat16),
            jax.ShapeDtypeStruct((T, V), jnp.float32),
            jax.ShapeDtypeStruct((_TCB, V), jnp.float32),
            jax.ShapeDtypeStruct((T, 16), jnp.float32),
            jax.ShapeDtypeStruct((2, _BSC, 32), jnp.int32),
        ],
    )(indices, tok_table, pos_t, W, b2d)


def _make_sc_gather(V, T):
    info = plsc.get_sparse_core_info()
    NC, NS = info.num_cores, info.num_subcores
    HT = T // 2
    CP = 32
    BW = _BSC // (NC * NS // 2)
    NCH = BW
    n_lane = V // 16

    mesh = plsc.VectorSubcoreMesh(core_axis_name="c", subcore_axis_name="s")

    @functools.partial(
        pl.kernel,
        mesh=mesh,
        out_type=jax.ShapeDtypeStruct((_BSC * T, V), jnp.float32),
        scratch_types=[
            pltpu.VMEM((NCH, CP), jnp.int32),
            pltpu.VMEM((CP, V), jnp.float32),
            pltpu.VMEM((HT, V), jnp.float32),
            pltpu.VMEM((HT, 16), jnp.float32),
            pltpu.SemaphoreType.DMA,
            pltpu.SemaphoreType.DMA,
        ],
        compiler_params=pltpu.CompilerParams(use_tc_tiling_on_sc=False),
    )
    def gather_kernel(idx_hbm, e_hbm, p_hbm, pt_hbm, out_hbm,
                      idx_v, buf0, p_v, pt_v, g0, s0):
        wid = lax.axis_index("s") * NC + lax.axis_index("c")
        wb = wid // 2
        wt = wid % 2
        pltpu.sync_copy(idx_hbm.at[wt, pl.ds(wb * BW, BW), :], idx_v)
        pltpu.sync_copy(p_hbm.at[pl.ds(wt * HT, HT), :], p_v)
        pltpu.sync_copy(pt_hbm.at[pl.ds(wt * HT, HT), :], pt_v)

        def gather(j):
            return pltpu.make_async_copy(
                e_hbm.at[idx_v.at[j, :]], buf0, g0)

        def scatter(j):
            row0 = (wb * BW + j) * T + wt * HT
            return pltpu.make_async_copy(
                buf0.at[pl.ds(0, HT), :],
                out_hbm.at[pl.ds(row0, HT), :], s0)

        def add_p(buf):
            @plsc.parallel_loop(0, HT, unroll=2)
            def _(i):
                for j in range(n_lane):
                    sl = pl.ds(j * 16, 16)
                    buf[i, sl] = buf[i, sl] + p_v[i, sl]
                sl = pl.ds(V - 16, 16)
                buf[i, sl] = buf[i, sl] + pt_v[i, :]

        def body(k, carry):
            gather(k).start()
            gather(k).wait()
            add_p(buf0)
            scatter(k).start()
            scatter(k).wait()
            return carry

        lax.fori_loop(0, NCH, body, 0)

    return gather_kernel


def _onehot_body(idx_ref, ebf_ref, pexp_ref, out_ref):
    idxv = idx_ref[0]
    V = ebf_ref.shape[0]
    iot = lax.broadcasted_iota(jnp.int32, (V, _TCB), 0)
    oht = (iot == idxv).astype(jnp.bfloat16)
    out_ref[...] = lax.dot_general(
        oht, ebf_ref[...], (((0,), (0,)), ((), ())),
        preferred_element_type=jnp.float32) + pexp_ref[...]


def _tc_onehot(idx2d, Ebf, Pexp, BT, V):
    n_sc_blocks = _BSC * 50 // _TCB
    n_blocks = BT // _TCB - n_sc_blocks
    return pl.pallas_call(
        _onehot_body,
        grid=(n_blocks,),
        in_specs=[
            pl.BlockSpec((1, 1, _TCB), lambda g: (g + n_sc_blocks, 0, 0)),
            pl.BlockSpec((V, V), lambda g: (0, 0)),
            pl.BlockSpec((_TCB, V), lambda g: (0, 0)),
        ],
        out_specs=pl.BlockSpec((_TCB, V), lambda g: (g + n_sc_blocks, 0)),
        out_shape=jax.ShapeDtypeStruct((BT, V), jnp.float32),
    )(idx2d, Ebf, Pexp)


def kernel(indices, tok_table, pos_table, W, b):
    Bsz, T = indices.shape
    V = W.shape[0]
    BT = Bsz * T

    idx32 = indices.astype(jnp.int32)
    E, Ebf, P, Pexp, PT, IDX3 = _precompute(idx32, tok_table,
                                            pos_table[:T], W,
                                            b.reshape(1, V))
    sc_out = _make_sc_gather(V, T)(IDX3, E, P, PT)
    idx2d = idx32.reshape(BT // _TCB, 1, _TCB)
    tc_out = _tc_onehot(idx2d, Ebf, Pexp, BT, V)
    out = lax.dynamic_update_slice(tc_out, sc_out, (0, 0))
    return out.reshape(Bsz, T, V)
